# Initial kernel scaffold; baseline (speedup 1.0000x reference)
import jax
import jax.numpy as jnp
from jax.experimental import pallas as pl


def kernel(outputs, targets, latent, raw):
    raise NotImplementedError("write your pallas kernel here")


# speedup vs baseline: 83.0641x; 83.0641x over previous
import functools

import jax
import jax.numpy as jnp
from jax import lax
from jax.experimental import pallas as pl
from jax.experimental.pallas import tpu as pltpu

_K = 25
_LAMBDA = 0.1
_PITERS = 50


def _recon_kernel(o_ref, t_ref, acc_ref):
    i = pl.program_id(0)

    @pl.when(i == 0)
    def _init():
        acc_ref[...] = jnp.zeros((1, 1), jnp.float32)

    d = o_ref[...] - t_ref[...]
    acc_ref[...] += jnp.sum(d * d).reshape(1, 1)


def _recon(outputs, targets):
    B, D = outputs.shape
    S = 512
    out = pl.pallas_call(
        _recon_kernel,
        grid=(B // S,),
        in_specs=[
            pl.BlockSpec((S, D), lambda i: (i, 0)),
            pl.BlockSpec((S, D), lambda i: (i, 0)),
        ],
        out_specs=pl.BlockSpec((1, 1), lambda i: (0, 0)),
        out_shape=jax.ShapeDtypeStruct((1, 1), jnp.float32),
    )(outputs, targets)
    return out[0, 0] / (B * D)


def _power_top(G, iters=_PITERS):
    B, K, _ = G.shape
    w = jnp.ones((B, K), G.dtype)

    def body(_, w):
        w = jnp.einsum("bkl,bl->bk", G, w)
        return w * lax.rsqrt(jnp.sum(w * w, axis=1, keepdims=True) + 1e-30)

    return lax.fori_loop(0, iters, body, w)


def kernel(outputs, targets, latent, raw):
    B, D = raw.shape
    recon = _recon(outputs, targets)

    sq = jnp.sum(raw * raw, axis=1)
    d2 = sq[:, None] + sq[None, :] - 2.0 * (raw @ raw.T)
    d2 = jnp.where(jnp.eye(B, dtype=bool), jnp.inf, d2)
    _, nbr = lax.top_k(-d2, _K)

    Z = latent[nbr]
    X = raw[nbr]
    Zc = Z - jnp.mean(Z, axis=1, keepdims=True)
    Xc = X - jnp.mean(X, axis=1, keepdims=True)
    Gz = jnp.einsum("bkd,bld->bkl", Zc, Zc)
    Gx = jnp.einsum("bkd,bld->bkl", Xc, Xc)
    M = jnp.einsum("bkd,bld->bkl", Zc, Xc)

    wz = _power_top(Gz)
    wx = _power_top(Gx)
    lz = jnp.einsum("bk,bkl,bl->b", wz, Gz, wz)
    lx = jnp.einsum("bk,bkl,bl->b", wx, Gx, wx)
    num = jnp.einsum("bk,bkl,bl->b", wz, M, wx)
    dot2 = (num * num) / (lz * lx + 1e-30)
    tsa = jnp.mean(2.0 - 2.0 * dot2)
    return recon + _LAMBDA * tsa


# speedup vs baseline: 1256.9192x; 15.1319x over previous
import functools

import jax
import jax.numpy as jnp
from jax import lax
from jax.experimental import pallas as pl
from jax.experimental.pallas import tpu as pltpu
from jax.experimental.pallas import tpu_sc as plsc

_B = 4096
_D = 64
_K = 25
_LAMBDA = 0.1
_PITERS = 20

_RB = 256
_SB = 512
_BIG = 1e30


def _knn_kernel(raw_ref, rows_ref, nbrt_ref, rawt_ref, d2_ref):
    i = pl.program_id(0)

    @pl.when(i == 0)
    def _prep():
        rawt_ref[...] = jnp.transpose(raw_ref[...], (1, 0))

    rawt = rawt_ref[...]
    rows = rows_ref[...]
    sq_full = jnp.sum(rawt * rawt, axis=0, keepdims=True)
    sq_rows = jnp.sum(rows * rows, axis=1, keepdims=True)
    dot = lax.dot_general(rows, rawt, (((1,), (0,)), ((), ())),
                          preferred_element_type=jnp.float32)
    d2 = sq_rows + sq_full - 2.0 * dot

    ci = lax.broadcasted_iota(jnp.int32, (_RB, _B), 1).astype(jnp.float32)
    ri = lax.broadcasted_iota(jnp.int32, (_RB, _B), 0).astype(jnp.float32)
    self_col = ri + jnp.float32(i * _RB)
    d2 = jnp.where(ci == self_col, _BIG, d2)
    d2_ref[...] = d2

    li = lax.broadcasted_iota(jnp.int32, (_RB, _K), 1).astype(jnp.float32)
    acc0 = jnp.zeros((_RB, _K), jnp.float32)

    def sel(k, acc):
        d2c = d2_ref[...]
        m = jnp.min(d2c, axis=1, keepdims=True)
        amf = jnp.min(jnp.where(d2c == m, ci, jnp.float32(2.0 * _B)),
                      axis=1, keepdims=True)
        d2_ref[...] = jnp.where(ci == amf, _BIG, d2c)
        return jnp.where(li == jnp.float32(k), amf, acc)

    acc = lax.fori_loop(0, _K, sel, acc0)
    nbrt_ref[...] = jnp.transpose(acc, (1, 0)).astype(jnp.int32)


def _knn(raw):
    return pl.pallas_call(
        _knn_kernel,
        grid=(_B // _RB,),
        in_specs=[
            pl.BlockSpec((_B, _D), lambda i: (0, 0)),
            pl.BlockSpec((_RB, _D), lambda i: (i, 0)),
        ],
        out_specs=pl.BlockSpec((_K, _RB), lambda i: (0, i)),
        out_shape=jax.ShapeDtypeStruct((_K, _B), jnp.int32),
        scratch_shapes=[
            pltpu.VMEM((_D, _B), jnp.float32),
            pltpu.VMEM((_RB, _B), jnp.float32),
        ],
    )(raw, raw)


_NW = 32
_ROWS_PER_W = _K * _B // _NW
_CH = 128
_NCH = _ROWS_PER_W // _CH


def _gather_sc(latent, raw, idxt_flat):
    mesh = plsc.VectorSubcoreMesh(core_axis_name="c", subcore_axis_name="s")

    @functools.partial(
        pl.kernel,
        mesh=mesh,
        compiler_params=pltpu.CompilerParams(use_tc_tiling_on_sc=False),
        out_type=[
            jax.ShapeDtypeStruct((_K * _B, _D), jnp.float32),
            jax.ShapeDtypeStruct((_K * _B, _D), jnp.float32),
        ],
        scratch_types=[
            pltpu.VMEM((_CH,), jnp.int32),
            pltpu.VMEM((_CH, _D), jnp.float32),
            pltpu.VMEM((_CH, _D), jnp.float32),
            pltpu.SemaphoreType.DMA,
            pltpu.SemaphoreType.DMA,
        ],
    )
    def k(lat_hbm, raw_hbm, idx_hbm, z_hbm, x_hbm, idx_v, zbuf, xbuf, s1, s2):
        wid = lax.axis_index("s") * 2 + lax.axis_index("c")
        base = wid * _ROWS_PER_W

        def chunk(c, carry):
            off = base + c * _CH
            pltpu.sync_copy(idx_hbm.at[pl.ds(off, _CH)], idx_v)
            pltpu.async_copy(lat_hbm.at[idx_v], zbuf, s1).wait()
            pltpu.sync_copy(zbuf, z_hbm.at[pl.ds(off, _CH)])
            pltpu.async_copy(raw_hbm.at[idx_v], xbuf, s2).wait()
            pltpu.sync_copy(xbuf, x_hbm.at[pl.ds(off, _CH)])
            return carry

        lax.fori_loop(0, _NCH, chunk, 0)

    return k(latent, raw, idxt_flat)


def _loss_kernel(zt_ref, xt_ref, o_ref, t_ref, out_ref):
    i = pl.program_id(0)

    @pl.when(i == 0)
    def _init():
        out_ref[...] = jnp.zeros((1, 1), jnp.float32)

    od = o_ref[...] - t_ref[...]
    recon_part = jnp.sum(od * od)

    def prep(ref):
        slabs = [jnp.transpose(ref[k], (1, 0)) for k in range(_K)]
        m = slabs[0]
        for k in range(1, _K):
            m = m + slabs[k]
        m = m * jnp.float32(1.0 / _K)
        return [s - m for s in slabs]

    def power_image(cs):
        w0 = jnp.ones((_K, _SB), jnp.float32)

        def it(_, w):
            y = cs[0] * w[0:1, :]
            for k in range(1, _K):
                y = y + cs[k] * w[k : k + 1, :]
            nw = [jnp.sum(cs[k] * y, axis=0, keepdims=True) for k in range(_K)]
            w2 = jnp.concatenate(nw, axis=0)
            ss = jnp.sum(w2 * w2, axis=0, keepdims=True)
            return w2 * lax.rsqrt(ss + 1e-30)

        w = lax.fori_loop(0, _PITERS, it, w0)
        y = cs[0] * w[0:1, :]
        for k in range(1, _K):
            y = y + cs[k] * w[k : k + 1, :]
        return y

    yz = power_image(prep(zt_ref))
    yx = power_image(prep(xt_ref))
    num = jnp.sum(yz * yx, axis=0, keepdims=True)
    lz = jnp.sum(yz * yz, axis=0, keepdims=True)
    lx = jnp.sum(yx * yx, axis=0, keepdims=True)
    dot2 = (num * num) / (lz * lx + 1e-30)
    tsa_part = 2.0 * _SB - 2.0 * jnp.sum(dot2)

    out_ref[...] += (recon_part / (_B * _D)
                     + (_LAMBDA / _B) * tsa_part).reshape(1, 1)


def _loss(zt, xt, outputs, targets):
    out = pl.pallas_call(
        _loss_kernel,
        grid=(_B // _SB,),
        in_specs=[
            pl.BlockSpec((_K, _SB, _D), lambda i: (0, i, 0)),
            pl.BlockSpec((_K, _SB, _D), lambda i: (0, i, 0)),
            pl.BlockSpec((_SB, _D), lambda i: (i, 0)),
            pl.BlockSpec((_SB, _D), lambda i: (i, 0)),
        ],
        out_specs=pl.BlockSpec((1, 1), lambda i: (0, 0)),
        out_shape=jax.ShapeDtypeStruct((1, 1), jnp.float32),
    )(zt, xt, outputs, targets)
    return out[0, 0]


def kernel(outputs, targets, latent, raw):
    nbrt = _knn(raw)
    zflat, xflat = _gather_sc(latent, raw, nbrt.reshape(_K * _B))
    zt = zflat.reshape(_K, _B, _D)
    xt = xflat.reshape(_K, _B, _D)
    return _loss(zt, xt, outputs, targets)


# speedup vs baseline: 1563.7009x; 1.2441x over previous
import functools

import jax
import jax.numpy as jnp
from jax import lax
from jax.experimental import pallas as pl
from jax.experimental.pallas import tpu as pltpu
from jax.experimental.pallas import tpu_sc as plsc

_B = 4096
_D = 64
_K = 25
_LAMBDA = 0.1
_PITERS = 12

_RB = 256
_SB = 512
_BIG = 1e30


def _knn_kernel(raw_ref, rows_ref, nbrt_ref, rawt_ref, key_ref):
    i = pl.program_id(0)

    @pl.when(i == 0)
    def _prep():
        rawt_ref[...] = jnp.transpose(raw_ref[...], (1, 0))

    rawt = rawt_ref[...]
    rows = rows_ref[...]
    sq_full = jnp.sum(rawt * rawt, axis=0, keepdims=True)
    sq_rows = jnp.sum(rows * rows, axis=1, keepdims=True)
    dot = lax.dot_general(rows, rawt, (((1,), (0,)), ((), ())),
                          preferred_element_type=jnp.float32)
    d2 = sq_rows + sq_full - 2.0 * dot

    ci = lax.broadcasted_iota(jnp.int32, (_RB, _B), 1)
    ri = lax.broadcasted_iota(jnp.int32, (_RB, _B), 0)
    self_col = ri + i * _RB
    bits = lax.bitcast_convert_type(jnp.maximum(d2, 0.0), jnp.int32)
    key = jnp.where(ci == self_col, jnp.int32(0x7FFFFFFF),
                    (bits & jnp.int32(~0xFFF)) | ci)
    key_ref[...] = key

    li = lax.broadcasted_iota(jnp.int32, (_RB, _K), 1)
    acc0 = jnp.zeros((_RB, _K), jnp.int32)

    def sel(k, carry):
        prev, acc = carry
        keyc = key_ref[...]
        keyc = jnp.where(keyc == prev, jnp.int32(0x7FFFFFFF), keyc)
        key_ref[...] = keyc
        kmin = jnp.min(keyc, axis=1, keepdims=True)
        acc = jnp.where(li == k, kmin & jnp.int32(0xFFF), acc)
        return (kmin, acc)

    _, acc = lax.fori_loop(0, _K, sel,
                           (jnp.full((_RB, 1), -1, jnp.int32), acc0))
    nbrt_ref[...] = jnp.transpose(acc, (1, 0))


def _knn(raw):
    return pl.pallas_call(
        _knn_kernel,
        grid=(_B // _RB,),
        in_specs=[
            pl.BlockSpec((_B, _D), lambda i: (0, 0)),
            pl.BlockSpec((_RB, _D), lambda i: (i, 0)),
        ],
        out_specs=pl.BlockSpec((_K, _RB), lambda i: (0, i)),
        out_shape=jax.ShapeDtypeStruct((_K, _B), jnp.int32),
        scratch_shapes=[
            pltpu.VMEM((_D, _B), jnp.float32),
            pltpu.VMEM((_RB, _B), jnp.int32),
        ],
    )(raw, raw)


_NW = 32
_ROWS_PER_W = _K * _B // _NW
_CH = 128
_NCH = _ROWS_PER_W // _CH


def _gather_sc(latent, raw, idxt_flat):
    mesh = plsc.VectorSubcoreMesh(core_axis_name="c", subcore_axis_name="s")

    @functools.partial(
        pl.kernel,
        mesh=mesh,
        compiler_params=pltpu.CompilerParams(use_tc_tiling_on_sc=False),
        out_type=[
            jax.ShapeDtypeStruct((_K * _B, _D), jnp.float32),
            jax.ShapeDtypeStruct((_K * _B, _D), jnp.float32),
        ],
        scratch_types=[
            pltpu.VMEM((_CH,), jnp.int32),
            pltpu.VMEM((_CH, _D), jnp.float32),
            pltpu.VMEM((_CH, _D), jnp.float32),
            pltpu.SemaphoreType.DMA,
            pltpu.SemaphoreType.DMA,
        ],
    )
    def k(lat_hbm, raw_hbm, idx_hbm, z_hbm, x_hbm, idx_v, zbuf, xbuf, s1, s2):
        wid = lax.axis_index("s") * 2 + lax.axis_index("c")
        base = wid * _ROWS_PER_W

        def chunk(c, carry):
            off = base + c * _CH
            pltpu.sync_copy(idx_hbm.at[pl.ds(off, _CH)], idx_v)
            cz = pltpu.async_copy(lat_hbm.at[idx_v], zbuf, s1)
            cx = pltpu.async_copy(raw_hbm.at[idx_v], xbuf, s2)
            cz.wait()
            cx.wait()
            pltpu.sync_copy(zbuf, z_hbm.at[pl.ds(off, _CH)])
            pltpu.sync_copy(xbuf, x_hbm.at[pl.ds(off, _CH)])
            return carry

        lax.fori_loop(0, _NCH, chunk, 0)

    return k(latent, raw, idxt_flat)


def _loss_kernel(zt_ref, xt_ref, o_ref, t_ref, out_ref):
    i = pl.program_id(0)

    @pl.when(i == 0)
    def _init():
        out_ref[...] = jnp.zeros((1, 1), jnp.float32)

    od = o_ref[...] - t_ref[...]
    recon_part = jnp.sum(od * od)

    def prep(ref):
        slabs = [jnp.transpose(ref[k], (1, 0)) for k in range(_K)]
        m = slabs[0]
        for k in range(1, _K):
            m = m + slabs[k]
        m = m * jnp.float32(1.0 / _K)
        return [s - m for s in slabs]

    def power_image(cs):
        w0 = jnp.ones((_K, _SB), jnp.float32)

        def it(_, w):
            y = cs[0] * w[0:1, :]
            for k in range(1, _K):
                y = y + cs[k] * w[k : k + 1, :]
            nw = [jnp.sum(cs[k] * y, axis=0, keepdims=True) for k in range(_K)]
            w2 = jnp.concatenate(nw, axis=0)
            ss = jnp.sum(w2 * w2, axis=0, keepdims=True)
            return w2 * lax.rsqrt(ss + 1e-30)

        w = lax.fori_loop(0, _PITERS, it, w0)
        y = cs[0] * w[0:1, :]
        for k in range(1, _K):
            y = y + cs[k] * w[k : k + 1, :]
        return y

    yz = power_image(prep(zt_ref))
    yx = power_image(prep(xt_ref))
    num = jnp.sum(yz * yx, axis=0, keepdims=True)
    lz = jnp.sum(yz * yz, axis=0, keepdims=True)
    lx = jnp.sum(yx * yx, axis=0, keepdims=True)
    dot2 = (num * num) / (lz * lx + 1e-30)
    tsa_part = 2.0 * _SB - 2.0 * jnp.sum(dot2)

    out_ref[...] += (recon_part / (_B * _D)
                     + (_LAMBDA / _B) * tsa_part).reshape(1, 1)


def _loss(zt, xt, outputs, targets):
    out = pl.pallas_call(
        _loss_kernel,
        grid=(_B // _SB,),
        in_specs=[
            pl.BlockSpec((_K, _SB, _D), lambda i: (0, i, 0)),
            pl.BlockSpec((_K, _SB, _D), lambda i: (0, i, 0)),
            pl.BlockSpec((_SB, _D), lambda i: (i, 0)),
            pl.BlockSpec((_SB, _D), lambda i: (i, 0)),
        ],
        out_specs=pl.BlockSpec((1, 1), lambda i: (0, 0)),
        out_shape=jax.ShapeDtypeStruct((1, 1), jnp.float32),
    )(zt, xt, outputs, targets)
    return out[0, 0]


def kernel(outputs, targets, latent, raw):
    nbrt = _knn(raw)
    zflat, xflat = _gather_sc(latent, raw, nbrt.reshape(_K * _B))
    zt = zflat.reshape(_K, _B, _D)
    xt = xflat.reshape(_K, _B, _D)
    return _loss(zt, xt, outputs, targets)


# speedup vs baseline: 2254.7054x; 1.4419x over previous
import functools

import jax
import jax.numpy as jnp
from jax import lax
from jax.experimental import pallas as pl
from jax.experimental.pallas import tpu as pltpu
from jax.experimental.pallas import tpu_sc as plsc

_B = 4096
_D = 64
_K = 25
_LAMBDA = 0.1
_PITERS = 12

_RB = 256
_SB = 512
_BIG = 1e30


_NCHUNK = 32
_CROWS = _B // _NCHUNK
_T = 8
_MAXI = 0x7FFFFFFF


def _knn_kernel(raw_ref, rows_ref, nbrt_ref, key_ref, stk_ref):
    i = pl.program_id(0)

    full = raw_ref[...]
    rows_t = jnp.transpose(rows_ref[...], (1, 0))
    sq_full = jnp.sum(full * full, axis=1, keepdims=True)
    sq_rows = jnp.sum(rows_t * rows_t, axis=0, keepdims=True)
    dot = lax.dot_general(full, rows_t, (((1,), (0,)), ((), ())),
                          preferred_element_type=jnp.float32)
    d2 = sq_full + sq_rows - 2.0 * dot

    ri = lax.broadcasted_iota(jnp.int32, (_B, _RB), 0)
    ci = lax.broadcasted_iota(jnp.int32, (_B, _RB), 1)
    bits = lax.bitcast_convert_type(jnp.maximum(d2, 0.0), jnp.int32)
    key = jnp.where(ri == ci + i * _RB, jnp.int32(_MAXI),
                    (bits & jnp.int32(~0xFFF)) | ri)
    key_ref[...] = key

    def peel(c, carry):
        blk = key_ref[pl.ds(c * _CROWS, _CROWS), :]
        ms = []
        for _t in range(_T):
            m = jnp.min(blk, axis=0, keepdims=True)
            ms.append(m)
            blk = jnp.where(blk == m, jnp.int32(_MAXI), blk)
        stk_ref[:, pl.ds(c, 1), :] = jnp.concatenate(ms, axis=0)[:, None, :]
        return carry

    lax.fori_loop(0, _NCHUNK, peel, 0)

    li = lax.broadcasted_iota(jnp.int32, (_K, _RB), 0)
    cnt0 = jnp.zeros((_NCHUNK, _RB), jnp.int32)
    acc0 = jnp.zeros((_K, _RB), jnp.int32)

    def sel(k, carry):
        cnt, acc = carry
        cur = jnp.full((_NCHUNK, _RB), jnp.int32(_MAXI), jnp.int32)
        for t in range(_T - 1, -1, -1):
            cur = jnp.where(cnt == t, stk_ref[t], cur)
        kmin = jnp.min(cur, axis=0, keepdims=True)
        cnt = cnt + (cur == kmin).astype(jnp.int32)
        acc = jnp.where(li == k, kmin & jnp.int32(0xFFF), acc)
        return (cnt, acc)

    _, acc = lax.fori_loop(0, _K, sel, (cnt0, acc0))
    nbrt_ref[...] = acc


def _knn(raw):
    return pl.pallas_call(
        _knn_kernel,
        grid=(_B // _RB,),
        in_specs=[
            pl.BlockSpec((_B, _D), lambda i: (0, 0)),
            pl.BlockSpec((_RB, _D), lambda i: (i, 0)),
        ],
        out_specs=pl.BlockSpec((_K, _RB), lambda i: (0, i)),
        out_shape=jax.ShapeDtypeStruct((_K, _B), jnp.int32),
        scratch_shapes=[
            pltpu.VMEM((_B, _RB), jnp.int32),
            pltpu.VMEM((_T, _NCHUNK, _RB), jnp.int32),
        ],
    )(raw, raw)


_NW = 32
_ROWS_PER_W = _K * _B // _NW
_CH = 128
_NCH = _ROWS_PER_W // _CH


def _gather_sc(latent, raw, idxt_flat):
    mesh = plsc.VectorSubcoreMesh(core_axis_name="c", subcore_axis_name="s")

    @functools.partial(
        pl.kernel,
        mesh=mesh,
        compiler_params=pltpu.CompilerParams(use_tc_tiling_on_sc=False),
        out_type=[
            jax.ShapeDtypeStruct((_K * _B, _D), jnp.float32),
            jax.ShapeDtypeStruct((_K * _B, _D), jnp.float32),
        ],
        scratch_types=[
            pltpu.VMEM((_CH,), jnp.int32),
            pltpu.VMEM((_CH, _D), jnp.float32),
            pltpu.VMEM((_CH, _D), jnp.float32),
            pltpu.SemaphoreType.DMA,
            pltpu.SemaphoreType.DMA,
        ],
    )
    def k(lat_hbm, raw_hbm, idx_hbm, z_hbm, x_hbm, idx_v, zbuf, xbuf, s1, s2):
        wid = lax.axis_index("s") * 2 + lax.axis_index("c")
        base = wid * _ROWS_PER_W

        def chunk(c, carry):
            off = base + c * _CH
            pltpu.sync_copy(idx_hbm.at[pl.ds(off, _CH)], idx_v)
            cz = pltpu.async_copy(lat_hbm.at[idx_v], zbuf, s1)
            cx = pltpu.async_copy(raw_hbm.at[idx_v], xbuf, s2)
            cz.wait()
            cx.wait()
            pltpu.sync_copy(zbuf, z_hbm.at[pl.ds(off, _CH)])
            pltpu.sync_copy(xbuf, x_hbm.at[pl.ds(off, _CH)])
            return carry

        lax.fori_loop(0, _NCH, chunk, 0)

    return k(latent, raw, idxt_flat)


def _loss_kernel(zt_ref, xt_ref, o_ref, t_ref, out_ref):
    i = pl.program_id(0)

    @pl.when(i == 0)
    def _init():
        out_ref[...] = jnp.zeros((1, 1), jnp.float32)

    od = o_ref[...] - t_ref[...]
    recon_part = jnp.sum(od * od)

    def prep(ref):
        slabs = [jnp.transpose(ref[k], (1, 0)) for k in range(_K)]
        m = slabs[0]
        for k in range(1, _K):
            m = m + slabs[k]
        m = m * jnp.float32(1.0 / _K)
        return [s - m for s in slabs]

    def power_image(cs):
        w0 = jnp.ones((_K, _SB), jnp.float32)

        def it(_, w):
            y = cs[0] * w[0:1, :]
            for k in range(1, _K):
                y = y + cs[k] * w[k : k + 1, :]
            nw = [jnp.sum(cs[k] * y, axis=0, keepdims=True) for k in range(_K)]
            w2 = jnp.concatenate(nw, axis=0)
            ss = jnp.sum(w2 * w2, axis=0, keepdims=True)
            return w2 * lax.rsqrt(ss + 1e-30)

        w = lax.fori_loop(0, _PITERS, it, w0)
        y = cs[0] * w[0:1, :]
        for k in range(1, _K):
            y = y + cs[k] * w[k : k + 1, :]
        return y

    yz = power_image(prep(zt_ref))
    yx = power_image(prep(xt_ref))
    num = jnp.sum(yz * yx, axis=0, keepdims=True)
    lz = jnp.sum(yz * yz, axis=0, keepdims=True)
    lx = jnp.sum(yx * yx, axis=0, keepdims=True)
    dot2 = (num * num) / (lz * lx + 1e-30)
    tsa_part = 2.0 * _SB - 2.0 * jnp.sum(dot2)

    out_ref[...] += (recon_part / (_B * _D)
                     + (_LAMBDA / _B) * tsa_part).reshape(1, 1)


def _loss(zt, xt, outputs, targets):
    out = pl.pallas_call(
        _loss_kernel,
        grid=(_B // _SB,),
        in_specs=[
            pl.BlockSpec((_K, _SB, _D), lambda i: (0, i, 0)),
            pl.BlockSpec((_K, _SB, _D), lambda i: (0, i, 0)),
            pl.BlockSpec((_SB, _D), lambda i: (i, 0)),
            pl.BlockSpec((_SB, _D), lambda i: (i, 0)),
        ],
        out_specs=pl.BlockSpec((1, 1), lambda i: (0, 0)),
        out_shape=jax.ShapeDtypeStruct((1, 1), jnp.float32),
    )(zt, xt, outputs, targets)
    return out[0, 0]


def kernel(outputs, targets, latent, raw):
    nbrt = _knn(raw)
    zflat, xflat = _gather_sc(latent, raw, nbrt.reshape(_K * _B))
    zt = zflat.reshape(_K, _B, _D)
    xt = xflat.reshape(_K, _B, _D)
    return _loss(zt, xt, outputs, targets)


# speedup vs baseline: 2463.5982x; 1.0926x over previous
import functools

import jax
import jax.numpy as jnp
from jax import lax
from jax.experimental import pallas as pl
from jax.experimental.pallas import tpu as pltpu
from jax.experimental.pallas import tpu_sc as plsc

_B = 4096
_D = 64
_K = 25
_LAMBDA = 0.1
_PITERS = 12

_HB = 2048
_RB = 256
_SB = 512
_BIG = 1e30


_NCHUNK = 32
_CROWS = _B // _NCHUNK
_T = 8
_MAXI = 0x7FFFFFFF


def _knn_kernel(h, raw_ref, rows_ref, nbrt_ref, key_ref, stk_ref):
    i = pl.program_id(0)

    full = raw_ref[...]
    rows_t = jnp.transpose(rows_ref[...], (1, 0))
    sq_full = jnp.sum(full * full, axis=1, keepdims=True)
    sq_rows = jnp.sum(rows_t * rows_t, axis=0, keepdims=True)
    dot = lax.dot_general(full, rows_t, (((1,), (0,)), ((), ())),
                          preferred_element_type=jnp.float32)
    d2 = sq_full + sq_rows - 2.0 * dot

    ri = lax.broadcasted_iota(jnp.int32, (_B, _RB), 0)
    ci = lax.broadcasted_iota(jnp.int32, (_B, _RB), 1)
    bits = lax.bitcast_convert_type(jnp.maximum(d2, 0.0), jnp.int32)
    key = jnp.where(ri == ci + i * _RB + h * _HB, jnp.int32(_MAXI),
                    (bits & jnp.int32(~0xFFF)) | ri)
    key_ref[...] = key

    def peel(c, carry):
        blk = key_ref[pl.ds(c * _CROWS, _CROWS), :]
        ms = []
        for _t in range(_T):
            m = jnp.min(blk, axis=0, keepdims=True)
            ms.append(m)
            blk = jnp.where(blk == m, jnp.int32(_MAXI), blk)
        stk_ref[:, pl.ds(c, 1), :] = jnp.concatenate(ms, axis=0)[:, None, :]
        return carry

    lax.fori_loop(0, _NCHUNK, peel, 0)

    li = lax.broadcasted_iota(jnp.int32, (_K, _RB), 0)
    cnt0 = jnp.zeros((_NCHUNK, _RB), jnp.int32)
    acc0 = jnp.zeros((_K, _RB), jnp.int32)

    def sel(k, carry):
        cnt, acc = carry
        cur = jnp.full((_NCHUNK, _RB), jnp.int32(_MAXI), jnp.int32)
        for t in range(_T - 1, -1, -1):
            cur = jnp.where(cnt == t, stk_ref[t], cur)
        kmin = jnp.min(cur, axis=0, keepdims=True)
        cnt = cnt + (cur == kmin).astype(jnp.int32)
        acc = jnp.where(li == k, kmin & jnp.int32(0xFFF), acc)
        return (cnt, acc)

    _, acc = lax.fori_loop(0, _K, sel, (cnt0, acc0))
    nbrt_ref[...] = acc


def _knn(raw, h):
    return pl.pallas_call(
        functools.partial(_knn_kernel, h),
        grid=(_HB // _RB,),
        in_specs=[
            pl.BlockSpec((_B, _D), lambda i: (0, 0)),
            pl.BlockSpec((_RB, _D), lambda i, _h=h: (i + _h * (_HB // _RB), 0)),
        ],
        out_specs=pl.BlockSpec((_K, _RB), lambda i: (0, i)),
        out_shape=jax.ShapeDtypeStruct((_K, _HB), jnp.int32),
        scratch_shapes=[
            pltpu.VMEM((_B, _RB), jnp.int32),
            pltpu.VMEM((_T, _NCHUNK, _RB), jnp.int32),
        ],
    )(raw, raw)


_NW = 32
_ROWS_PER_W = _K * _HB // _NW
_CH = 80
_NCH = _ROWS_PER_W // _CH


def _gather_sc(latent, raw, idxt_flat):
    mesh = plsc.VectorSubcoreMesh(core_axis_name="c", subcore_axis_name="s")

    @functools.partial(
        pl.kernel,
        mesh=mesh,
        compiler_params=pltpu.CompilerParams(use_tc_tiling_on_sc=False),
        out_type=[
            jax.ShapeDtypeStruct((_K * _HB, _D), jnp.float32),
            jax.ShapeDtypeStruct((_K * _HB, _D), jnp.float32),
        ],
        scratch_types=[
            pltpu.VMEM((_CH,), jnp.int32),
            pltpu.VMEM((_CH, _D), jnp.float32),
            pltpu.VMEM((_CH, _D), jnp.float32),
            pltpu.SemaphoreType.DMA,
            pltpu.SemaphoreType.DMA,
        ],
    )
    def k(lat_hbm, raw_hbm, idx_hbm, z_hbm, x_hbm, idx_v, zbuf, xbuf, s1, s2):
        wid = lax.axis_index("s") * 2 + lax.axis_index("c")
        base = wid * _ROWS_PER_W

        def chunk(c, carry):
            off = base + c * _CH
            pltpu.sync_copy(idx_hbm.at[pl.ds(off, _CH)], idx_v)
            cz = pltpu.async_copy(lat_hbm.at[idx_v], zbuf, s1)
            cx = pltpu.async_copy(raw_hbm.at[idx_v], xbuf, s2)
            cz.wait()
            cx.wait()
            pltpu.sync_copy(zbuf, z_hbm.at[pl.ds(off, _CH)])
            pltpu.sync_copy(xbuf, x_hbm.at[pl.ds(off, _CH)])
            return carry

        lax.fori_loop(0, _NCH, chunk, 0)

    return k(latent, raw, idxt_flat)


def _loss_kernel(zt_ref, xt_ref, o_ref, t_ref, out_ref):
    i = pl.program_id(0)

    @pl.when(i == 0)
    def _init():
        out_ref[...] = jnp.zeros((1, 1), jnp.float32)

    od = o_ref[...] - t_ref[...]
    recon_part = jnp.sum(od * od)

    def prep(ref):
        slabs = [jnp.transpose(ref[k], (1, 0)) for k in range(_K)]
        m = slabs[0]
        for k in range(1, _K):
            m = m + slabs[k]
        m = m * jnp.float32(1.0 / _K)
        return [s - m for s in slabs]

    def power_image(cs):
        w0 = jnp.ones((_K, _SB), jnp.float32)

        def it(_, w):
            y = cs[0] * w[0:1, :]
            for k in range(1, _K):
                y = y + cs[k] * w[k : k + 1, :]
            nw = [jnp.sum(cs[k] * y, axis=0, keepdims=True) for k in range(_K)]
            w2 = jnp.concatenate(nw, axis=0)
            ss = jnp.sum(w2 * w2, axis=0, keepdims=True)
            return w2 * lax.rsqrt(ss + 1e-30)

        w = lax.fori_loop(0, _PITERS, it, w0)
        y = cs[0] * w[0:1, :]
        for k in range(1, _K):
            y = y + cs[k] * w[k : k + 1, :]
        return y

    yz = power_image(prep(zt_ref))
    yx = power_image(prep(xt_ref))
    num = jnp.sum(yz * yx, axis=0, keepdims=True)
    lz = jnp.sum(yz * yz, axis=0, keepdims=True)
    lx = jnp.sum(yx * yx, axis=0, keepdims=True)
    dot2 = (num * num) / (lz * lx + 1e-30)
    tsa_part = 2.0 * _SB - 2.0 * jnp.sum(dot2)

    out_ref[...] += (recon_part / (_B * _D)
                     + (_LAMBDA / _B) * tsa_part).reshape(1, 1)


def _loss(zt, xt, outputs, targets):
    out = pl.pallas_call(
        _loss_kernel,
        grid=(_HB // _SB,),
        in_specs=[
            pl.BlockSpec((_K, _SB, _D), lambda i: (0, i, 0)),
            pl.BlockSpec((_K, _SB, _D), lambda i: (0, i, 0)),
            pl.BlockSpec((_SB, _D), lambda i: (i, 0)),
            pl.BlockSpec((_SB, _D), lambda i: (i, 0)),
        ],
        out_specs=pl.BlockSpec((1, 1), lambda i: (0, 0)),
        out_shape=jax.ShapeDtypeStruct((1, 1), jnp.float32),
    )(zt, xt, outputs, targets)
    return out[0, 0]


def kernel(outputs, targets, latent, raw):
    total = None
    for h in range(2):
        nbrt = _knn(raw, h)
        zflat, xflat = _gather_sc(latent, raw, nbrt.reshape(_K * _HB))
        zt = zflat.reshape(_K, _HB, _D)
        xt = xflat.reshape(_K, _HB, _D)
        part = _loss(zt, xt,
                     outputs[h * _HB:(h + 1) * _HB],
                     targets[h * _HB:(h + 1) * _HB])
        total = part if total is None else total + part
    return total


# speedup vs baseline: 2705.2989x; 1.0981x over previous
import functools

import jax
import jax.numpy as jnp
from jax import lax
from jax.experimental import pallas as pl
from jax.experimental.pallas import tpu as pltpu
from jax.experimental.pallas import tpu_sc as plsc

_B = 4096
_D = 64
_K = 25
_LAMBDA = 0.1
_PITERS = 8

_HB = 2048
_RB = 256
_SB = 512
_BIG = 1e30


_NCHUNK = 32
_CROWS = _B // _NCHUNK
_T = 8
_MAXI = 0x7FFFFFFF


def _knn_kernel(h, raw_ref, rows_ref, nbrt_ref, key_ref, stk_ref):
    i = pl.program_id(0)

    full = raw_ref[...]
    rows_t = jnp.transpose(rows_ref[...], (1, 0))
    sq_full = jnp.sum(full * full, axis=1, keepdims=True)
    sq_rows = jnp.sum(rows_t * rows_t, axis=0, keepdims=True)
    dot = lax.dot_general(full, rows_t, (((1,), (0,)), ((), ())),
                          preferred_element_type=jnp.float32)
    d2 = sq_full + sq_rows - 2.0 * dot

    ri = lax.broadcasted_iota(jnp.int32, (_B, _RB), 0)
    ci = lax.broadcasted_iota(jnp.int32, (_B, _RB), 1)
    bits = lax.bitcast_convert_type(jnp.maximum(d2, 0.0), jnp.int32)
    key = jnp.where(ri == ci + i * _RB + h * _HB, jnp.int32(_MAXI),
                    (bits & jnp.int32(~0xFFF)) | ri)
    key_ref[...] = key

    def peel(c, carry):
        blk = key_ref[pl.ds(c * _CROWS, _CROWS), :]
        ms = []
        for _t in range(_T):
            m = jnp.min(blk, axis=0, keepdims=True)
            ms.append(m)
            blk = jnp.where(blk == m, jnp.int32(_MAXI), blk)
        stk_ref[:, pl.ds(c, 1), :] = jnp.concatenate(ms, axis=0)[:, None, :]
        return carry

    lax.fori_loop(0, _NCHUNK, peel, 0)

    li = lax.broadcasted_iota(jnp.int32, (_K, _RB), 0)
    cnt0 = jnp.zeros((_NCHUNK, _RB), jnp.int32)
    acc0 = jnp.zeros((_K, _RB), jnp.int32)

    def sel(k, carry):
        cnt, acc = carry
        cur = jnp.full((_NCHUNK, _RB), jnp.int32(_MAXI), jnp.int32)
        for t in range(_T - 1, -1, -1):
            cur = jnp.where(cnt == t, stk_ref[t], cur)
        kmin = jnp.min(cur, axis=0, keepdims=True)
        cnt = cnt + (cur == kmin).astype(jnp.int32)
        acc = jnp.where(li == k, kmin & jnp.int32(0xFFF), acc)
        return (cnt, acc)

    _, acc = lax.fori_loop(0, _K, sel, (cnt0, acc0))
    nbrt_ref[...] = acc


def _knn(raw, h):
    return pl.pallas_call(
        functools.partial(_knn_kernel, h),
        grid=(_HB // _RB,),
        in_specs=[
            pl.BlockSpec((_B, _D), lambda i: (0, 0)),
            pl.BlockSpec((_RB, _D), lambda i, _h=h: (i + _h * (_HB // _RB), 0)),
        ],
        out_specs=pl.BlockSpec((_K, _RB), lambda i: (0, i)),
        out_shape=jax.ShapeDtypeStruct((_K, _HB), jnp.int32),
        scratch_shapes=[
            pltpu.VMEM((_B, _RB), jnp.int32),
            pltpu.VMEM((_T, _NCHUNK, _RB), jnp.int32),
        ],
    )(raw, raw)


_NW = 32
_ROWS_PER_W = _K * _HB // _NW
_CH = 80
_NCH = _ROWS_PER_W // _CH


def _gather_sc(latent, raw, idxt_flat):
    mesh = plsc.VectorSubcoreMesh(core_axis_name="c", subcore_axis_name="s")

    @functools.partial(
        pl.kernel,
        mesh=mesh,
        compiler_params=pltpu.CompilerParams(use_tc_tiling_on_sc=False),
        out_type=[
            jax.ShapeDtypeStruct((_K * _HB, _D), jnp.float32),
            jax.ShapeDtypeStruct((_K * _HB, _D), jnp.float32),
        ],
        scratch_types=[
            pltpu.VMEM((_CH,), jnp.int32),
            pltpu.VMEM((_CH, _D), jnp.float32),
            pltpu.VMEM((_CH, _D), jnp.float32),
            pltpu.SemaphoreType.DMA,
            pltpu.SemaphoreType.DMA,
        ],
    )
    def k(lat_hbm, raw_hbm, idx_hbm, z_hbm, x_hbm, idx_v, zbuf, xbuf, s1, s2):
        wid = lax.axis_index("s") * 2 + lax.axis_index("c")
        base = wid * _ROWS_PER_W

        def chunk(c, carry):
            off = base + c * _CH
            pltpu.sync_copy(idx_hbm.at[pl.ds(off, _CH)], idx_v)
            cz = pltpu.async_copy(lat_hbm.at[idx_v], zbuf, s1)
            cx = pltpu.async_copy(raw_hbm.at[idx_v], xbuf, s2)
            cz.wait()
            cx.wait()
            pltpu.sync_copy(zbuf, z_hbm.at[pl.ds(off, _CH)])
            pltpu.sync_copy(xbuf, x_hbm.at[pl.ds(off, _CH)])
            return carry

        lax.fori_loop(0, _NCH, chunk, 0)

    return k(latent, raw, idxt_flat)


def _loss_kernel(zt_ref, xt_ref, o_ref, t_ref, out_ref):
    i = pl.program_id(0)

    @pl.when(i == 0)
    def _init():
        out_ref[...] = jnp.zeros((1, 1), jnp.float32)

    od = o_ref[...] - t_ref[...]
    recon_part = jnp.sum(od * od)

    def prep(ref):
        slabs = [jnp.transpose(ref[k], (1, 0)) for k in range(_K)]
        m = slabs[0]
        for k in range(1, _K):
            m = m + slabs[k]
        m = m * jnp.float32(1.0 / _K)
        return [s - m for s in slabs]

    def power_image(cs):
        w0 = jnp.ones((_K, _SB), jnp.float32)

        def it(_, w):
            y = cs[0] * w[0:1, :]
            for k in range(1, _K):
                y = y + cs[k] * w[k : k + 1, :]
            nw = [jnp.sum(cs[k] * y, axis=0, keepdims=True) for k in range(_K)]
            w2 = jnp.concatenate(nw, axis=0)
            ss = jnp.sum(w2 * w2, axis=0, keepdims=True)
            return w2 * lax.rsqrt(ss + 1e-30)

        w = lax.fori_loop(0, _PITERS, it, w0)
        y = cs[0] * w[0:1, :]
        for k in range(1, _K):
            y = y + cs[k] * w[k : k + 1, :]
        return y

    yz = power_image(prep(zt_ref))
    yx = power_image(prep(xt_ref))
    num = jnp.sum(yz * yx, axis=0, keepdims=True)
    lz = jnp.sum(yz * yz, axis=0, keepdims=True)
    lx = jnp.sum(yx * yx, axis=0, keepdims=True)
    dot2 = (num * num) / (lz * lx + 1e-30)
    tsa_part = 2.0 * _SB - 2.0 * jnp.sum(dot2)

    out_ref[...] += (recon_part / (_B * _D)
                     + (_LAMBDA / _B) * tsa_part).reshape(1, 1)


def _loss(zt, xt, outputs, targets):
    out = pl.pallas_call(
        _loss_kernel,
        grid=(_HB // _SB,),
        in_specs=[
            pl.BlockSpec((_K, _SB, _D), lambda i: (0, i, 0)),
            pl.BlockSpec((_K, _SB, _D), lambda i: (0, i, 0)),
            pl.BlockSpec((_SB, _D), lambda i: (i, 0)),
            pl.BlockSpec((_SB, _D), lambda i: (i, 0)),
        ],
        out_specs=pl.BlockSpec((1, 1), lambda i: (0, 0)),
        out_shape=jax.ShapeDtypeStruct((1, 1), jnp.float32),
    )(zt, xt, outputs, targets)
    return out[0, 0]


def kernel(outputs, targets, latent, raw):
    total = None
    for h in range(2):
        nbrt = _knn(raw, h)
        zflat, xflat = _gather_sc(latent, raw, nbrt.reshape(_K * _HB))
        zt = zflat.reshape(_K, _HB, _D)
        xt = xflat.reshape(_K, _HB, _D)
        part = _loss(zt, xt,
                     outputs[h * _HB:(h + 1) * _HB],
                     targets[h * _HB:(h + 1) * _HB])
        total = part if total is None else total + part
    return total


# speedup vs baseline: 2960.9766x; 1.0945x over previous
import functools

import jax
import jax.numpy as jnp
from jax import lax
from jax.experimental import pallas as pl
from jax.experimental.pallas import tpu as pltpu
from jax.experimental.pallas import tpu_sc as plsc

_B = 4096
_D = 64
_K = 25
_LAMBDA = 0.1
_PITERS = 8

_HB = 2048
_RB = 256
_SB = 512
_BIG = 1e30


_NCHUNK = 32
_CROWS = _B // _NCHUNK
_T = 8
_MAXI = 0x7FFFFFFF


def _knn_kernel(h, raw_ref, rows_ref, nbrt_ref, key_ref, stk_ref):
    i = pl.program_id(0)

    full = raw_ref[...]
    rows_t = jnp.transpose(rows_ref[...], (1, 0))
    sq_full = jnp.sum(full * full, axis=1, keepdims=True)
    sq_rows = jnp.sum(rows_t * rows_t, axis=0, keepdims=True)
    dot = lax.dot_general(full, rows_t, (((1,), (0,)), ((), ())),
                          preferred_element_type=jnp.float32)
    d2 = sq_full + sq_rows - 2.0 * dot

    ri = lax.broadcasted_iota(jnp.int32, (_B, _RB), 0)
    ci = lax.broadcasted_iota(jnp.int32, (_B, _RB), 1)
    bits = lax.bitcast_convert_type(jnp.maximum(d2, 0.0), jnp.int32)
    key = jnp.where(ri == ci + i * _RB + h * _HB, jnp.int32(_MAXI),
                    (bits & jnp.int32(~0xFFF)) | ri)
    key_ref[...] = key

    def peel(c, carry):
        blk = key_ref[pl.ds(c * _CROWS, _CROWS), :]
        ms = []
        for _t in range(_T):
            m = jnp.min(blk, axis=0, keepdims=True)
            ms.append(m)
            blk = jnp.where(blk == m, jnp.int32(_MAXI), blk)
        stk_ref[:, pl.ds(c, 1), :] = jnp.concatenate(ms, axis=0)[:, None, :]
        return carry

    lax.fori_loop(0, _NCHUNK, peel, 0)

    li = lax.broadcasted_iota(jnp.int32, (_K, _RB), 0)
    cnt0 = jnp.zeros((_NCHUNK, _RB), jnp.int32)
    acc0 = jnp.zeros((_K, _RB), jnp.int32)

    def sel(k, carry):
        cnt, acc = carry
        cur = jnp.full((_NCHUNK, _RB), jnp.int32(_MAXI), jnp.int32)
        for t in range(_T - 1, -1, -1):
            cur = jnp.where(cnt == t, stk_ref[t], cur)
        kmin = jnp.min(cur, axis=0, keepdims=True)
        cnt = cnt + (cur == kmin).astype(jnp.int32)
        acc = jnp.where(li == k, kmin & jnp.int32(0xFFF), acc)
        return (cnt, acc)

    _, acc = lax.fori_loop(0, _K, sel, (cnt0, acc0))
    nbrt_ref[...] = acc


def _knn(raw, h):
    return pl.pallas_call(
        functools.partial(_knn_kernel, h),
        grid=(_HB // _RB,),
        in_specs=[
            pl.BlockSpec((_B, _D), lambda i: (0, 0)),
            pl.BlockSpec((_RB, _D), lambda i, _h=h: (i + _h * (_HB // _RB), 0)),
        ],
        out_specs=pl.BlockSpec((_K, _RB), lambda i: (0, i)),
        out_shape=jax.ShapeDtypeStruct((_K, _HB), jnp.int32),
        scratch_shapes=[
            pltpu.VMEM((_B, _RB), jnp.int32),
            pltpu.VMEM((_T, _NCHUNK, _RB), jnp.int32),
        ],
    )(raw, raw)


_NW = 32
_ROWS_PER_W = _K * _HB // _NW
_CH = 80
_NCH = _ROWS_PER_W // _CH


def _gather_sc(latent, raw, idxt_flat):
    mesh = plsc.VectorSubcoreMesh(core_axis_name="c", subcore_axis_name="s")

    @functools.partial(
        pl.kernel,
        mesh=mesh,
        compiler_params=pltpu.CompilerParams(use_tc_tiling_on_sc=False),
        out_type=[
            jax.ShapeDtypeStruct((_K * _HB, _D), jnp.float32),
            jax.ShapeDtypeStruct((_K * _HB, _D), jnp.float32),
        ],
        scratch_types=[
            pltpu.VMEM((_CH,), jnp.int32),
            pltpu.VMEM((_CH, _D), jnp.float32),
            pltpu.VMEM((_CH, _D), jnp.float32),
            pltpu.SemaphoreType.DMA,
            pltpu.SemaphoreType.DMA,
        ],
    )
    def k(lat_hbm, raw_hbm, idx_hbm, z_hbm, x_hbm, idx_v, zbuf, xbuf, s1, s2):
        wid = lax.axis_index("s") * 2 + lax.axis_index("c")
        base = wid * _ROWS_PER_W

        def chunk(c, carry):
            off = base + c * _CH
            pltpu.sync_copy(idx_hbm.at[pl.ds(off, _CH)], idx_v)
            cz = pltpu.async_copy(lat_hbm.at[idx_v], zbuf, s1)
            cx = pltpu.async_copy(raw_hbm.at[idx_v], xbuf, s2)
            cz.wait()
            cx.wait()
            pltpu.sync_copy(zbuf, z_hbm.at[pl.ds(off, _CH)])
            pltpu.sync_copy(xbuf, x_hbm.at[pl.ds(off, _CH)])
            return carry

        lax.fori_loop(0, _NCH, chunk, 0)

    return k(latent, raw, idxt_flat)


def _loss_kernel(zt_ref, xt_ref, o_ref, t_ref, out_ref):
    i = pl.program_id(0)

    @pl.when(i == 0)
    def _init():
        out_ref[...] = jnp.zeros((1, 1), jnp.float32)

    od = o_ref[...] - t_ref[...]
    recon_part = jnp.sum(od * od)

    def prep(ref):
        zs = [ref[k] for k in range(_K)]
        m = zs[0]
        for k in range(1, _K):
            m = m + zs[k]
        m = m * jnp.float32(1.0 / _K)
        return [jnp.transpose((z - m).astype(jnp.bfloat16), (1, 0))
                for z in zs]

    def power_image(cs):
        w0 = jnp.ones((_K, _SB), jnp.bfloat16)

        def it(_, w):
            y = cs[0] * w[0:1, :]
            for k in range(1, _K):
                y = y + cs[k] * w[k : k + 1, :]
            nw = [jnp.sum(cs[k] * y, axis=0, keepdims=True) for k in range(_K)]
            return jnp.concatenate(nw, axis=0)

        w = lax.fori_loop(0, _PITERS, it, w0)
        y = cs[0] * w[0:1, :]
        for k in range(1, _K):
            y = y + cs[k] * w[k : k + 1, :]
        y = y.astype(jnp.float32)
        scale = 1.0 / (jnp.max(jnp.abs(y), axis=0, keepdims=True) + 1e-30)
        return y * scale

    yz = power_image(prep(zt_ref))
    yx = power_image(prep(xt_ref))
    num = jnp.sum(yz * yx, axis=0, keepdims=True)
    lz = jnp.sum(yz * yz, axis=0, keepdims=True)
    lx = jnp.sum(yx * yx, axis=0, keepdims=True)
    dot2 = (num * num) / (lz * lx + 1e-30)
    tsa_part = 2.0 * _SB - 2.0 * jnp.sum(dot2)

    out_ref[...] += (recon_part / (_B * _D)
                     + (_LAMBDA / _B) * tsa_part).reshape(1, 1)


def _loss(zt, xt, outputs, targets):
    out = pl.pallas_call(
        _loss_kernel,
        grid=(_HB // _SB,),
        in_specs=[
            pl.BlockSpec((_K, _SB, _D), lambda i: (0, i, 0)),
            pl.BlockSpec((_K, _SB, _D), lambda i: (0, i, 0)),
            pl.BlockSpec((_SB, _D), lambda i: (i, 0)),
            pl.BlockSpec((_SB, _D), lambda i: (i, 0)),
        ],
        out_specs=pl.BlockSpec((1, 1), lambda i: (0, 0)),
        out_shape=jax.ShapeDtypeStruct((1, 1), jnp.float32),
    )(zt, xt, outputs, targets)
    return out[0, 0]


def kernel(outputs, targets, latent, raw):
    total = None
    for h in range(2):
        nbrt = _knn(raw, h)
        zflat, xflat = _gather_sc(latent, raw, nbrt.reshape(_K * _HB))
        zt = zflat.reshape(_K, _HB, _D)
        xt = xflat.reshape(_K, _HB, _D)
        part = _loss(zt, xt,
                     outputs[h * _HB:(h + 1) * _HB],
                     targets[h * _HB:(h + 1) * _HB])
        total = part if total is None else total + part
    return total
